# Initial kernel scaffold; baseline (speedup 1.0000x reference)
import jax
import jax.numpy as jnp
from jax.experimental import pallas as pl


def kernel(a, b):
    raise NotImplementedError("write your pallas kernel here")


# speedup vs baseline: 218.0627x; 218.0627x over previous
import functools

import jax
import jax.numpy as jnp
from jax.experimental import pallas as pl
from jax.experimental.pallas import tpu as pltpu

B = 4
N = 4096
D = 16
ROW_TILE = 512
NT = N // ROW_TILE
_INV = 1.0 / (B * N)


def _chamfer_kernel(a_ref, b_ref, out_ref, colmin_ref):
    bi = pl.program_id(0)
    ti = pl.program_id(1)

    x = a_ref[0]
    y = b_ref[0]

    zz = jax.lax.dot_general(
        x * -2.0, y,
        dimension_numbers=(((1,), (1,)), ((), ())),
        preferred_element_type=jnp.float32,
    )
    xx = jnp.sum(x * x, axis=1, keepdims=True)
    yy = jnp.sum(y * y, axis=1, keepdims=True).T
    p = zz + xx + yy

    row_min = jnp.min(p, axis=1)
    col_min = jnp.min(p, axis=0, keepdims=True)

    @pl.when(jnp.logical_and(bi == 0, ti == 0))
    def _init():
        out_ref[0, 0] = 0.0

    out_ref[0, 0] += jnp.sum(row_min) * _INV

    @pl.when(ti == 0)
    def _col_first():
        colmin_ref[...] = col_min

    @pl.when(ti != 0)
    def _col_rest():
        colmin_ref[...] = jnp.minimum(colmin_ref[...], col_min)

    @pl.when(ti == NT - 1)
    def _col_finish():
        out_ref[0, 0] += jnp.sum(colmin_ref[...]) * _INV


@jax.jit
def kernel(a, b):
    out = pl.pallas_call(
        _chamfer_kernel,
        grid=(B, NT),
        in_specs=[
            pl.BlockSpec((1, ROW_TILE, D), lambda bi, ti: (bi, ti, ti - ti)),
            pl.BlockSpec((1, N, D), lambda bi, ti: (bi, ti - ti, ti - ti)),
        ],
        out_specs=pl.BlockSpec(
            (1, 1), lambda bi, ti: (ti - ti, ti - ti), memory_space=pltpu.SMEM
        ),
        out_shape=jax.ShapeDtypeStruct((1, 1), jnp.float32),
        scratch_shapes=[pltpu.VMEM((1, N), jnp.float32)],
        compiler_params=pltpu.CompilerParams(
            dimension_semantics=("arbitrary", "arbitrary"),
        ),
    )(a, b)
    return out[0, 0]


# speedup vs baseline: 339.3406x; 1.5562x over previous
import functools

import jax
import jax.numpy as jnp
from jax.experimental import pallas as pl
from jax.experimental.pallas import tpu as pltpu

B = 4
N = 4096
D = 16
ROW_TILE = 512
NT = N // ROW_TILE
_INV = 1.0 / (B * N)


def _chamfer_kernel(a_ref, b_ref, out_ref, colmin_ref, yaug_ref):
    bi = pl.program_id(0)
    ti = pl.program_id(1)

    x = a_ref[0]

    @pl.when(ti == 0)
    def _build_yaug():
        y = b_ref[0]
        yy = jnp.sum(y * y, axis=1, keepdims=True)
        ones = jnp.ones((N, 1), jnp.float32)
        yaug_ref[...] = jnp.concatenate([y, ones, yy], axis=1)

    xx = jnp.sum(x * x, axis=1, keepdims=True)
    x_aug = jnp.concatenate(
        [x * -2.0, xx, jnp.ones((ROW_TILE, 1), jnp.float32)], axis=1
    )
    p = jax.lax.dot_general(
        x_aug, yaug_ref[...],
        dimension_numbers=(((1,), (1,)), ((), ())),
        preferred_element_type=jnp.float32,
    )

    row_min = jnp.min(p, axis=1)
    col_min = jnp.min(p, axis=0, keepdims=True)

    @pl.when(jnp.logical_and(bi == 0, ti == 0))
    def _init():
        out_ref[0, 0] = 0.0

    out_ref[0, 0] += jnp.sum(row_min) * _INV

    @pl.when(ti == 0)
    def _col_first():
        colmin_ref[...] = col_min

    @pl.when(ti != 0)
    def _col_rest():
        colmin_ref[...] = jnp.minimum(colmin_ref[...], col_min)

    @pl.when(ti == NT - 1)
    def _col_finish():
        out_ref[0, 0] += jnp.sum(colmin_ref[...]) * _INV


@jax.jit
def kernel(a, b):
    out = pl.pallas_call(
        _chamfer_kernel,
        grid=(B, NT),
        in_specs=[
            pl.BlockSpec((1, ROW_TILE, D), lambda bi, ti: (bi, ti, ti - ti)),
            pl.BlockSpec((1, N, D), lambda bi, ti: (bi, ti - ti, ti - ti)),
        ],
        out_specs=pl.BlockSpec(
            (1, 1), lambda bi, ti: (ti - ti, ti - ti), memory_space=pltpu.SMEM
        ),
        out_shape=jax.ShapeDtypeStruct((1, 1), jnp.float32),
        scratch_shapes=[
            pltpu.VMEM((1, N), jnp.float32),
            pltpu.VMEM((N, D + 2), jnp.float32),
        ],
        compiler_params=pltpu.CompilerParams(
            dimension_semantics=("arbitrary", "arbitrary"),
        ),
    )(a, b)
    return out[0, 0]


# speedup vs baseline: 381.6968x; 1.1248x over previous
import functools

import jax
import jax.numpy as jnp
from jax.experimental import pallas as pl
from jax.experimental.pallas import tpu as pltpu

B = 4
N = 4096
D = 16
ROW_TILE = 1024
NT = N // ROW_TILE
_INV = 1.0 / (B * N)


def _chamfer_kernel(a_ref, b_ref, out_ref, colmin_ref, yaug_ref):
    bi = pl.program_id(0)
    ti = pl.program_id(1)

    x = a_ref[0]

    @pl.when(ti == 0)
    def _build_yaug():
        y = b_ref[0]
        yy = jnp.sum(y * y, axis=1, keepdims=True)
        ones = jnp.ones((N, 1), jnp.float32)
        yaug_ref[...] = jnp.concatenate([y, ones, yy], axis=1)

    xx = jnp.sum(x * x, axis=1, keepdims=True)
    x_aug = jnp.concatenate(
        [x * -2.0, xx, jnp.ones((ROW_TILE, 1), jnp.float32)], axis=1
    )
    p = jax.lax.dot_general(
        x_aug, yaug_ref[...],
        dimension_numbers=(((1,), (1,)), ((), ())),
        preferred_element_type=jnp.float32,
    )

    row_min = jnp.min(p, axis=1)
    col_min = jnp.min(p, axis=0, keepdims=True)

    @pl.when(jnp.logical_and(bi == 0, ti == 0))
    def _init():
        out_ref[0, 0] = 0.0

    out_ref[0, 0] += jnp.sum(row_min) * _INV

    @pl.when(ti == 0)
    def _col_first():
        colmin_ref[...] = col_min

    @pl.when(ti != 0)
    def _col_rest():
        colmin_ref[...] = jnp.minimum(colmin_ref[...], col_min)

    @pl.when(ti == NT - 1)
    def _col_finish():
        out_ref[0, 0] += jnp.sum(colmin_ref[...]) * _INV


@jax.jit
def kernel(a, b):
    out = pl.pallas_call(
        _chamfer_kernel,
        grid=(B, NT),
        in_specs=[
            pl.BlockSpec((1, ROW_TILE, D), lambda bi, ti: (bi, ti, ti - ti)),
            pl.BlockSpec((1, N, D), lambda bi, ti: (bi, ti - ti, ti - ti)),
        ],
        out_specs=pl.BlockSpec(
            (1, 1), lambda bi, ti: (ti - ti, ti - ti), memory_space=pltpu.SMEM
        ),
        out_shape=jax.ShapeDtypeStruct((1, 1), jnp.float32),
        scratch_shapes=[
            pltpu.VMEM((1, N), jnp.float32),
            pltpu.VMEM((N, D + 2), jnp.float32),
        ],
        compiler_params=pltpu.CompilerParams(
            dimension_semantics=("arbitrary", "arbitrary"),
        ),
    )(a, b)
    return out[0, 0]


# speedup vs baseline: 406.0656x; 1.0638x over previous
import functools

import jax
import jax.numpy as jnp
from jax.experimental import pallas as pl
from jax.experimental.pallas import tpu as pltpu

B = 4
N = 4096
D = 16
ROW_TILE = 2048
NT = N // ROW_TILE
_INV = 1.0 / (B * N)


def _chamfer_kernel(a_ref, b_ref, out_ref, colmin_ref, yaug_ref):
    bi = pl.program_id(0)
    ti = pl.program_id(1)

    x = a_ref[0]

    @pl.when(ti == 0)
    def _build_yaug():
        y = b_ref[0]
        yy = jnp.sum(y * y, axis=1, keepdims=True)
        ones = jnp.ones((N, 1), jnp.float32)
        yaug_ref[...] = jnp.concatenate([y, ones, yy], axis=1)

    xx = jnp.sum(x * x, axis=1, keepdims=True)
    x_aug = jnp.concatenate(
        [x * -2.0, xx, jnp.ones((ROW_TILE, 1), jnp.float32)], axis=1
    )
    p = jax.lax.dot_general(
        x_aug, yaug_ref[...],
        dimension_numbers=(((1,), (1,)), ((), ())),
        preferred_element_type=jnp.float32,
    )

    row_min = jnp.min(p, axis=1)
    col_min = jnp.min(p, axis=0, keepdims=True)

    @pl.when(jnp.logical_and(bi == 0, ti == 0))
    def _init():
        out_ref[0, 0] = 0.0

    out_ref[0, 0] += jnp.sum(row_min) * _INV

    @pl.when(ti == 0)
    def _col_first():
        colmin_ref[...] = col_min

    @pl.when(ti != 0)
    def _col_rest():
        colmin_ref[...] = jnp.minimum(colmin_ref[...], col_min)

    @pl.when(ti == NT - 1)
    def _col_finish():
        out_ref[0, 0] += jnp.sum(colmin_ref[...]) * _INV


@jax.jit
def kernel(a, b):
    out = pl.pallas_call(
        _chamfer_kernel,
        grid=(B, NT),
        in_specs=[
            pl.BlockSpec((1, ROW_TILE, D), lambda bi, ti: (bi, ti, ti - ti)),
            pl.BlockSpec((1, N, D), lambda bi, ti: (bi, ti - ti, ti - ti)),
        ],
        out_specs=pl.BlockSpec(
            (1, 1), lambda bi, ti: (ti - ti, ti - ti), memory_space=pltpu.SMEM
        ),
        out_shape=jax.ShapeDtypeStruct((1, 1), jnp.float32),
        scratch_shapes=[
            pltpu.VMEM((1, N), jnp.float32),
            pltpu.VMEM((N, D + 2), jnp.float32),
        ],
        compiler_params=pltpu.CompilerParams(
            dimension_semantics=("arbitrary", "arbitrary"),
        ),
    )(a, b)
    return out[0, 0]
